# baseline (speedup 1.0000x reference)
import functools

import jax
import jax.numpy as jnp
from jax import lax
from jax.experimental import pallas as pl
from jax.experimental.pallas import tpu as pltpu
from jax.experimental.pallas import tpu_sc as plsc


def _table_body(cb_ref, Wpo_ref, bpo_ref, Wu_ref, bu_ref, Tc_ref):
    tmp = jnp.dot(cb_ref[...], Wpo_ref[...],
                  preferred_element_type=jnp.float32) + bpo_ref[...]
    T = jnp.dot(tmp, Wu_ref[...],
                preferred_element_type=jnp.float32) + bu_ref[...]
    Tc_ref[...] = jnp.clip(T, -1.0, 1.0)


def _assign_body(x_ref, Wd_ref, bd_ref, Wp_ref, bp_ref, cbT_ref, c2_ref,
                 idx_ref):
    h = jnp.dot(x_ref[...], Wd_ref[...],
                preferred_element_type=jnp.float32) + bd_ref[...]
    z = jnp.dot(h, Wp_ref[...],
                preferred_element_type=jnp.float32) + bp_ref[...]
    z2 = jnp.sum(z * z, axis=-1, keepdims=True)
    cross = jnp.dot(z, cbT_ref[...],
                    preferred_element_type=jnp.float32)
    dist = z2 - 2.0 * cross + c2_ref[...]
    idx_ref[0, 0, :] = jnp.argmin(dist, axis=1).astype(jnp.int32)


_SC_CORES = 2
_SC_SUBCORES = 16
_SC_WORKERS = _SC_CORES * _SC_SUBCORES


def _onehot_body(idx_ref, Tc_ref, out_ref):
    blk = out_ref.shape[0]
    k = Tc_ref.shape[0]
    idx = idx_ref[0, 0, :]
    iota = lax.broadcasted_iota(jnp.int32, (blk, k), 1)
    oh = (iota == idx[:, None]).astype(jnp.float32)
    out_ref[...] = jnp.dot(oh, Tc_ref[...], preferred_element_type=jnp.float32)


def _sc_gather_call(table, idx, out_dim):
    tok = idx.shape[0]
    K = table.shape[0]
    bpw = tok // _SC_WORKERS
    chunk = min(bpw, 32)
    nch = bpw // chunk
    mesh = plsc.VectorSubcoreMesh(core_axis_name="c", subcore_axis_name="s",
                                  num_cores=_SC_CORES,
                                  num_subcores=_SC_SUBCORES)

    @functools.partial(
        pl.kernel,
        out_type=jax.ShapeDtypeStruct((tok, out_dim), jnp.float32),
        mesh=mesh,
        scratch_types=[
            pltpu.VMEM((bpw,), jnp.int32),
            pltpu.VMEM((2, chunk, out_dim), jnp.float32),
            pltpu.SemaphoreType.DMA,
            (pltpu.SemaphoreType.DMA, pltpu.SemaphoreType.DMA),
        ],
    )
    def gather(table_hbm, idx_hbm, out_hbm, idx_v, rows_v, idx_sem, sems):
        cid = lax.axis_index("c")
        sid = lax.axis_index("s")
        wid = sid * _SC_CORES + cid
        base = wid * bpw
        pltpu.async_copy(idx_hbm.at[pl.ds(base, bpw)], idx_v, idx_sem).wait()
        pltpu.async_copy(
            table_hbm.at[idx_v.at[pl.ds(0, chunk)]], rows_v.at[0], sems[0])
        for c in range(nch):
            nxt = c + 1
            if nxt < nch:
                pltpu.async_copy(
                    table_hbm.at[idx_v.at[pl.ds(nxt * chunk, chunk)]],
                    rows_v.at[nxt % 2], sems[nxt % 2])
            pltpu.make_async_copy(
                table_hbm.at[idx_v.at[pl.ds(c * chunk, chunk)]],
                rows_v.at[c % 2], sems[c % 2]).wait()
            pltpu.sync_copy(rows_v.at[c % 2],
                            out_hbm.at[pl.ds(base + c * chunk, chunk)])

    return gather(table, idx)


def kernel(x, W_down, b_down, W_pin, b_pin, codebook, W_pout, b_pout, W_up,
           b_up):
    B, S, IN = x.shape
    H = W_down.shape[1]
    CD = W_pin.shape[1]
    K = codebook.shape[0]
    OUT = W_up.shape[1]
    tok = B * S

    x2d = x.reshape(tok, IN)
    cbT = codebook.T
    c2 = jnp.sum(codebook * codebook, axis=-1).reshape(1, K)

    Tc = pl.pallas_call(
        _table_body,
        out_shape=jax.ShapeDtypeStruct((K, OUT), jnp.float32),
    )(codebook, W_pout, b_pout.reshape(1, H), W_up, b_up.reshape(1, OUT))

    blk = 2048
    nb = tok // blk
    idx3 = pl.pallas_call(
        _assign_body,
        grid=(nb,),
        in_specs=[
            pl.BlockSpec((blk, IN), lambda i: (i, 0)),
            pl.BlockSpec((IN, H), lambda i: (0, 0)),
            pl.BlockSpec((1, H), lambda i: (0, 0)),
            pl.BlockSpec((H, CD), lambda i: (0, 0)),
            pl.BlockSpec((1, CD), lambda i: (0, 0)),
            pl.BlockSpec((CD, K), lambda i: (0, 0)),
            pl.BlockSpec((1, K), lambda i: (0, 0)),
        ],
        out_specs=pl.BlockSpec((1, 1, blk), lambda i: (i, 0, 0)),
        out_shape=jax.ShapeDtypeStruct((nb, 1, blk), jnp.int32),
    )(x2d, W_down, b_down.reshape(1, H), W_pin, b_pin.reshape(1, CD), cbT, c2)

    idx = idx3.reshape(tok)

    sc_tok = tok // 8
    sc_out = _sc_gather_call(Tc, idx[:sc_tok], OUT)

    tc_first = sc_tok // blk
    tc_out = pl.pallas_call(
        _onehot_body,
        grid=(nb - tc_first,),
        in_specs=[
            pl.BlockSpec((1, 1, blk), lambda i: (i + tc_first, 0, 0)),
            pl.BlockSpec((K, OUT), lambda i: (0, 0)),
        ],
        out_specs=pl.BlockSpec((blk, OUT), lambda i: (i + tc_first, 0)),
        out_shape=jax.ShapeDtypeStruct((tok, OUT), jnp.float32),
    )(idx3, Tc)

    out2d = lax.dynamic_update_slice(tc_out, sc_out, (0, 0))
    out = out2d.reshape(B, S, OUT)
    indices = idx.reshape(B, S)
    commit_loss = jnp.zeros((), dtype=jnp.float32)
    return out, indices, commit_loss


# speedup vs baseline: 1.0825x; 1.0825x over previous
import functools

import jax
import jax.numpy as jnp
from jax import lax
from jax.experimental import pallas as pl
from jax.experimental.pallas import tpu as pltpu
from jax.experimental.pallas import tpu_sc as plsc


def _table_body(cb_ref, Wpo_ref, bpo_ref, Wu_ref, bu_ref, Tc_ref):
    tmp = jnp.dot(cb_ref[...], Wpo_ref[...],
                  preferred_element_type=jnp.float32) + bpo_ref[...]
    T = jnp.dot(tmp, Wu_ref[...],
                preferred_element_type=jnp.float32) + bu_ref[...]
    Tc_ref[...] = jnp.clip(T, -1.0, 1.0)


def _assign_idx(x_ref, Wd_ref, bd_ref, Wp_ref, bp_ref, cbT_ref, c2_ref):
    h = jnp.dot(x_ref[...], Wd_ref[...],
                preferred_element_type=jnp.float32) + bd_ref[...]
    z = jnp.dot(h, Wp_ref[...],
                preferred_element_type=jnp.float32) + bp_ref[...]
    z2 = jnp.sum(z * z, axis=-1, keepdims=True)
    cross = jnp.dot(z, cbT_ref[...],
                    preferred_element_type=jnp.float32)
    dist = z2 - 2.0 * cross + c2_ref[...]
    return jnp.argmin(dist, axis=1).astype(jnp.int32)


def _assign_body(x_ref, Wd_ref, bd_ref, Wp_ref, bp_ref, cbT_ref, c2_ref,
                 idx_ref):
    idx_ref[0, 0, :] = _assign_idx(x_ref, Wd_ref, bd_ref, Wp_ref, bp_ref,
                                   cbT_ref, c2_ref)


def _assign_onehot_body(x_ref, Wd_ref, bd_ref, Wp_ref, bp_ref, cbT_ref,
                        c2_ref, Tc_ref, idx_ref, out_ref):
    idx = _assign_idx(x_ref, Wd_ref, bd_ref, Wp_ref, bp_ref, cbT_ref, c2_ref)
    idx_ref[0, 0, :] = idx
    blk = out_ref.shape[0]
    k = Tc_ref.shape[0]
    iota = lax.broadcasted_iota(jnp.int32, (blk, k), 1)
    oh = (iota == idx[:, None]).astype(jnp.float32)
    out_ref[...] = jnp.dot(oh, Tc_ref[...], preferred_element_type=jnp.float32)


_SC_CORES = 2
_SC_SUBCORES = 16
_SC_WORKERS = _SC_CORES * _SC_SUBCORES


def _onehot_body(idx_ref, Tc_ref, out_ref):
    blk = out_ref.shape[0]
    k = Tc_ref.shape[0]
    idx = idx_ref[0, 0, :]
    iota = lax.broadcasted_iota(jnp.int32, (blk, k), 1)
    oh = (iota == idx[:, None]).astype(jnp.float32)
    out_ref[...] = jnp.dot(oh, Tc_ref[...], preferred_element_type=jnp.float32)


def _sc_gather_call(table, idx, out_dim):
    tok = idx.shape[0]
    K = table.shape[0]
    bpw = tok // _SC_WORKERS
    chunk = min(bpw, 32)
    nch = bpw // chunk
    mesh = plsc.VectorSubcoreMesh(core_axis_name="c", subcore_axis_name="s",
                                  num_cores=_SC_CORES,
                                  num_subcores=_SC_SUBCORES)

    @functools.partial(
        pl.kernel,
        out_type=jax.ShapeDtypeStruct((tok, out_dim), jnp.float32),
        mesh=mesh,
        scratch_types=[
            pltpu.VMEM((bpw,), jnp.int32),
            pltpu.VMEM((2, chunk, out_dim), jnp.float32),
            pltpu.SemaphoreType.DMA,
            (pltpu.SemaphoreType.DMA, pltpu.SemaphoreType.DMA),
        ],
    )
    def gather(table_hbm, idx_hbm, out_hbm, idx_v, rows_v, idx_sem, sems):
        cid = lax.axis_index("c")
        sid = lax.axis_index("s")
        wid = sid * _SC_CORES + cid
        base = wid * bpw
        pltpu.async_copy(idx_hbm.at[pl.ds(base, bpw)], idx_v, idx_sem).wait()
        pltpu.async_copy(
            table_hbm.at[idx_v.at[pl.ds(0, chunk)]], rows_v.at[0], sems[0])
        for c in range(nch):
            nxt = c + 1
            if nxt < nch:
                pltpu.async_copy(
                    table_hbm.at[idx_v.at[pl.ds(nxt * chunk, chunk)]],
                    rows_v.at[nxt % 2], sems[nxt % 2])
            pltpu.make_async_copy(
                table_hbm.at[idx_v.at[pl.ds(c * chunk, chunk)]],
                rows_v.at[c % 2], sems[c % 2]).wait()
            pltpu.sync_copy(rows_v.at[c % 2],
                            out_hbm.at[pl.ds(base + c * chunk, chunk)])

    return gather(table, idx)


def kernel(x, W_down, b_down, W_pin, b_pin, codebook, W_pout, b_pout, W_up,
           b_up):
    B, S, IN = x.shape
    H = W_down.shape[1]
    CD = W_pin.shape[1]
    K = codebook.shape[0]
    OUT = W_up.shape[1]
    tok = B * S

    x2d = x.reshape(tok, IN)
    cbT = codebook.T
    c2 = jnp.sum(codebook * codebook, axis=-1).reshape(1, K)

    Tc = pl.pallas_call(
        _table_body,
        out_shape=jax.ShapeDtypeStruct((K, OUT), jnp.float32),
    )(codebook, W_pout, b_pout.reshape(1, H), W_up, b_up.reshape(1, OUT))

    blk = 1024
    nb = tok // blk
    weight_specs = [
        pl.BlockSpec((IN, H), lambda i: (0, 0)),
        pl.BlockSpec((1, H), lambda i: (0, 0)),
        pl.BlockSpec((H, CD), lambda i: (0, 0)),
        pl.BlockSpec((1, CD), lambda i: (0, 0)),
        pl.BlockSpec((CD, K), lambda i: (0, 0)),
        pl.BlockSpec((1, K), lambda i: (0, 0)),
    ]
    weights = (W_down, b_down.reshape(1, H), W_pin, b_pin.reshape(1, CD),
               cbT, c2)

    idxA = pl.pallas_call(
        _assign_body,
        grid=(1,),
        in_specs=[pl.BlockSpec((blk, IN), lambda i: (i, 0))] + weight_specs,
        out_specs=pl.BlockSpec((1, 1, blk), lambda i: (i, 0, 0)),
        out_shape=jax.ShapeDtypeStruct((1, 1, blk), jnp.int32),
    )(x2d, *weights)

    sc_out = _sc_gather_call(Tc, idxA.reshape(blk), OUT)

    idxB, tc_out = pl.pallas_call(
        _assign_onehot_body,
        grid=(nb - 1,),
        in_specs=[pl.BlockSpec((blk, IN), lambda i: (i + 1, 0))]
        + weight_specs
        + [pl.BlockSpec((K, OUT), lambda i: (0, 0))],
        out_specs=[
            pl.BlockSpec((1, 1, blk), lambda i: (i, 0, 0)),
            pl.BlockSpec((blk, OUT), lambda i: (i + 1, 0)),
        ],
        out_shape=[
            jax.ShapeDtypeStruct((nb - 1, 1, blk), jnp.int32),
            jax.ShapeDtypeStruct((tok, OUT), jnp.float32),
        ],
    )(x2d, *weights, Tc)

    out2d = lax.dynamic_update_slice(tc_out, sc_out, (0, 0))
    out = out2d.reshape(B, S, OUT)
    idx = jnp.concatenate([idxA.reshape(blk),
                           idxB.reshape(tok - blk)])
    indices = idx.reshape(B, S)
    commit_loss = jnp.zeros((), dtype=jnp.float32)
    return out, indices, commit_loss
